# Initial kernel scaffold; baseline (speedup 1.0000x reference)
import jax
import jax.numpy as jnp
from jax.experimental import pallas as pl


def kernel(inputs, weight):
    raise NotImplementedError("write your pallas kernel here")


# speedup vs baseline: 2.1321x; 2.1321x over previous
import functools

import jax
import jax.numpy as jnp
from jax.experimental import pallas as pl
from jax.experimental.pallas import tpu as pltpu

_K = 8192
_D = 32
_N = 8192
_BLK = 256
_NBLK = _N // _BLK
_COMMIT = 0.25


def _vq_body(x_ref, w_ref,
             soft_ref, idx_ref, quant_ref, wn_ref,
             loss_ref, perp_ref, active_ref, usage_ref,
             hist_ref, acc_ref):
    i = pl.program_id(0)

    w = w_ref[...]
    wn = w / jnp.maximum(jnp.sqrt(jnp.sum(w * w, axis=1, keepdims=True)), 1e-12)

    @pl.when(i == 0)
    def _init():
        wn_ref[...] = wn
        hist_ref[...] = jnp.zeros_like(hist_ref)
        acc_ref[0, 0] = 0.0

    x = x_ref[...]
    fn = x / jnp.maximum(jnp.sqrt(jnp.sum(x * x, axis=1, keepdims=True)), 1e-12)

    dots = jax.lax.dot_general(fn, wn, (((1,), (1,)), ((), ())),
                               preferred_element_type=jnp.float32)
    d = 2.0 - 2.0 * dots

    dmin = jnp.min(d, axis=1, keepdims=True)
    e = jnp.exp(dmin - d)
    s = jnp.sum(e, axis=1, keepdims=True)
    soft_ref[...] = e / s

    cols = jax.lax.broadcasted_iota(jnp.int32, (_BLK, _K), 1)
    idx = jnp.min(jnp.where(d == dmin, cols, _K), axis=1).astype(jnp.int32)
    idx_ref[...] = idx[None, :]

    oh = (cols == idx[:, None]).astype(jnp.float32)
    quant = jax.lax.dot_general(oh, wn, (((1,), (0,)), ((), ())),
                                preferred_element_type=jnp.float32)
    quant_ref[...] = quant
    hist_ref[...] += jnp.sum(oh, axis=0, keepdims=True)
    acc_ref[0, 0] += jnp.sum((quant - fn) ** 2)

    @pl.when(i == _NBLK - 1)
    def _finish():
        hist = hist_ref[...]
        avg = hist / _N
        perp_ref[0, 0] = jnp.exp(-jnp.sum(avg * jnp.log(avg + 1e-10)))
        active = jnp.sum((hist > 0.0).astype(jnp.int32))
        active_ref[0, 0] = active
        usage_ref[0, 0] = active.astype(jnp.float32) / _K * 100.0
        loss_ref[0, 0] = _COMMIT * acc_ref[0, 0] / (_N * _D)


@jax.jit
def kernel(inputs, weight):
    flat = inputs.reshape(_N, _D)

    out_shapes = (
        jax.ShapeDtypeStruct((_N, _K), jnp.float32),
        jax.ShapeDtypeStruct((1, _N), jnp.int32),
        jax.ShapeDtypeStruct((_N, _D), jnp.float32),
        jax.ShapeDtypeStruct((_K, _D), jnp.float32),
        jax.ShapeDtypeStruct((1, 1), jnp.float32),
        jax.ShapeDtypeStruct((1, 1), jnp.float32),
        jax.ShapeDtypeStruct((1, 1), jnp.int32),
        jax.ShapeDtypeStruct((1, 1), jnp.float32),
    )
    out_specs = (
        pl.BlockSpec((_BLK, _K), lambda i: (i, 0)),
        pl.BlockSpec((1, _BLK), lambda i: (0, i)),
        pl.BlockSpec((_BLK, _D), lambda i: (i, 0)),
        pl.BlockSpec((_K, _D), lambda i: (0, 0)),
        pl.BlockSpec(memory_space=pltpu.SMEM),
        pl.BlockSpec(memory_space=pltpu.SMEM),
        pl.BlockSpec(memory_space=pltpu.SMEM),
        pl.BlockSpec(memory_space=pltpu.SMEM),
    )
    in_specs = [
        pl.BlockSpec((_BLK, _D), lambda i: (i, 0)),
        pl.BlockSpec((_K, _D), lambda i: (0, 0)),
    ]

    soft, idx, quant, _wn, loss, perp, active, usage = pl.pallas_call(
        _vq_body,
        grid=(_NBLK,),
        in_specs=in_specs,
        out_specs=out_specs,
        out_shape=out_shapes,
        scratch_shapes=[
            pltpu.VMEM((1, _K), jnp.float32),
            pltpu.SMEM((1, 1), jnp.float32),
        ],
        compiler_params=pltpu.CompilerParams(
            dimension_semantics=("arbitrary",),
        ),
    )(flat, weight)

    return (
        loss.reshape(()),
        quant.reshape(inputs.shape),
        soft,
        perp.reshape(()),
        idx.reshape(_N, 1),
        active.reshape(()).astype(jnp.int32),
        usage.reshape(()),
    )


# speedup vs baseline: 2.9886x; 1.4017x over previous
import functools

import jax
import jax.numpy as jnp
from jax.experimental import pallas as pl
from jax.experimental.pallas import tpu as pltpu

_K = 8192
_D = 32
_N = 8192
_BLK = 256
_NBLK = _N // _BLK
_COMMIT = 0.25


def _vq_body(x_ref, w_ref,
             soft_ref, idx_ref, quant_ref, wn_ref,
             loss_ref, perp_ref, active_ref, usage_ref,
             hist_ref, acc_ref, wm2_ref):
    i = pl.program_id(0)

    @pl.when(i == 0)
    def _init():
        w = w_ref[...]
        wn = w / jnp.maximum(jnp.sqrt(jnp.sum(w * w, axis=1, keepdims=True)),
                             1e-12)
        wn_ref[...] = wn
        wm2_ref[...] = wn * -2.0
        hist_ref[...] = jnp.zeros_like(hist_ref)
        acc_ref[0, 0] = 0.0

    x = x_ref[...]
    fn = x / jnp.maximum(jnp.sqrt(jnp.sum(x * x, axis=1, keepdims=True)), 1e-12)

    t = jax.lax.dot_general(fn, wm2_ref[...], (((1,), (1,)), ((), ())),
                            preferred_element_type=jnp.float32)
    d = 2.0 + t

    dmin = jnp.min(d, axis=1, keepdims=True)
    e = jnp.exp(dmin - d)
    s = jnp.sum(e, axis=1, keepdims=True)
    soft_ref[...] = e / s

    idx = jnp.argmin(d, axis=1).astype(jnp.int32)
    idx_ref[...] = idx[None, :]

    cols = jax.lax.broadcasted_iota(jnp.int32, (_BLK, _K), 1)
    oh = (cols == idx[:, None]).astype(jnp.float32)
    quant = jax.lax.dot_general(oh, wn_ref[...], (((1,), (0,)), ((), ())),
                                preferred_element_type=jnp.float32)
    quant_ref[...] = quant
    hist_ref[...] += jnp.sum(oh, axis=0, keepdims=True)
    acc_ref[0, 0] += jnp.sum((quant - fn) ** 2)

    @pl.when(i == _NBLK - 1)
    def _finish():
        hist = hist_ref[...]
        avg = hist / _N
        perp_ref[0, 0] = jnp.exp(-jnp.sum(avg * jnp.log(avg + 1e-10)))
        active = jnp.sum((hist > 0.0).astype(jnp.int32))
        active_ref[0, 0] = active
        usage_ref[0, 0] = active.astype(jnp.float32) / _K * 100.0
        loss_ref[0, 0] = _COMMIT * acc_ref[0, 0] / (_N * _D)


@jax.jit
def kernel(inputs, weight):
    flat = inputs.reshape(_N, _D)

    out_shapes = (
        jax.ShapeDtypeStruct((_N, _K), jnp.float32),
        jax.ShapeDtypeStruct((1, _N), jnp.int32),
        jax.ShapeDtypeStruct((_N, _D), jnp.float32),
        jax.ShapeDtypeStruct((_K, _D), jnp.float32),
        jax.ShapeDtypeStruct((1, 1), jnp.float32),
        jax.ShapeDtypeStruct((1, 1), jnp.float32),
        jax.ShapeDtypeStruct((1, 1), jnp.int32),
        jax.ShapeDtypeStruct((1, 1), jnp.float32),
    )
    out_specs = (
        pl.BlockSpec((_BLK, _K), lambda i: (i, 0)),
        pl.BlockSpec((1, _BLK), lambda i: (0, i)),
        pl.BlockSpec((_BLK, _D), lambda i: (i, 0)),
        pl.BlockSpec((_K, _D), lambda i: (0, 0)),
        pl.BlockSpec(memory_space=pltpu.SMEM),
        pl.BlockSpec(memory_space=pltpu.SMEM),
        pl.BlockSpec(memory_space=pltpu.SMEM),
        pl.BlockSpec(memory_space=pltpu.SMEM),
    )
    in_specs = [
        pl.BlockSpec((_BLK, _D), lambda i: (i, 0)),
        pl.BlockSpec((_K, _D), lambda i: (0, 0)),
    ]

    soft, idx, quant, _wn, loss, perp, active, usage = pl.pallas_call(
        _vq_body,
        grid=(_NBLK,),
        in_specs=in_specs,
        out_specs=out_specs,
        out_shape=out_shapes,
        scratch_shapes=[
            pltpu.VMEM((1, _K), jnp.float32),
            pltpu.SMEM((1, 1), jnp.float32),
            pltpu.VMEM((_K, _D), jnp.float32),
        ],
        compiler_params=pltpu.CompilerParams(
            dimension_semantics=("arbitrary",),
        ),
    )(flat, weight)

    return (
        loss.reshape(()),
        quant.reshape(inputs.shape),
        soft,
        perp.reshape(()),
        idx.reshape(_N, 1),
        active.reshape(()).astype(jnp.int32),
        usage.reshape(()),
    )


# speedup vs baseline: 3.3063x; 1.1063x over previous
import functools

import jax
import jax.numpy as jnp
from jax import lax
from jax.experimental import pallas as pl
from jax.experimental.pallas import tpu as pltpu
from jax.experimental.pallas import tpu_sc as plsc

_K = 8192
_D = 32
_N = 8192
_BLK = 256
_NBLK = _N // _BLK
_COMMIT = 0.25

_SC_NC = 2
_SC_NS = 16
_NW = _SC_NC * _SC_NS
_BPW = _N // _NW
_CHUNK = 128
_NCHUNK = _BPW // _CHUNK


def _vq_main_body(x_ref, w_ref,
                  soft_ref, idx_ref, wn_ref,
                  loss_ref, perp_ref, active_ref, usage_ref,
                  acc_ref, wm2_ref, hist_ref):
    i = pl.program_id(0)

    @pl.when(i == 0)
    def _init():
        w = w_ref[...]
        wn = w / jnp.maximum(jnp.sqrt(jnp.sum(w * w, axis=1, keepdims=True)),
                             1e-12)
        wn_ref[...] = jnp.concatenate(
            [wn, jnp.zeros((_K, 128 - _D), jnp.float32)], axis=1)
        wm2_ref[...] = wn * -2.0
        hist_ref[...] = jnp.zeros_like(hist_ref)
        acc_ref[0, 0] = 0.0

    x = x_ref[...]
    fn = x / jnp.maximum(jnp.sqrt(jnp.sum(x * x, axis=1, keepdims=True)), 1e-12)

    t = jax.lax.dot_general(fn, wm2_ref[...], (((1,), (1,)), ((), ())),
                            preferred_element_type=jnp.float32)
    d = 2.0 + t

    dmin = jnp.min(d, axis=1, keepdims=True)
    e = jnp.exp(dmin - d)
    s = jnp.sum(e, axis=1, keepdims=True)
    soft_ref[...] = e / s

    idx = jnp.argmin(d, axis=1).astype(jnp.int32)
    idx_ref[...] = idx[None, :]

    cols = jax.lax.broadcasted_iota(jnp.int32, (_BLK, _K), 1)
    oh = (cols == idx[:, None]).astype(jnp.float32)
    hist_ref[...] += jnp.sum(oh, axis=0, keepdims=True)
    acc_ref[0, 0] += jnp.sum(dmin)

    @pl.when(i == _NBLK - 1)
    def _finish():
        hist = hist_ref[...]
        avg = hist / _N
        perp_ref[0, 0] = jnp.exp(-jnp.sum(avg * jnp.log(avg + 1e-10)))
        active = jnp.sum((hist > 0.0).astype(jnp.int32))
        active_ref[0, 0] = active
        usage_ref[0, 0] = active.astype(jnp.float32) / _K * 100.0
        loss_ref[0, 0] = _COMMIT * acc_ref[0, 0] / (_N * _D)


def _sc_gather_body(wn_hbm, idx_hbm, quant_hbm, idx_v, rows_v, sem):
    c = lax.axis_index("c")
    s = lax.axis_index("s")
    wid = s * _SC_NC + c

    pltpu.sync_copy(idx_hbm.at[pl.ds(wid * _NCHUNK, _NCHUNK)], idx_v)
    for j in range(_NCHUNK):
        pltpu.async_copy(wn_hbm.at[idx_v.at[j]],
                         rows_v.at[pl.ds(j * _CHUNK, _CHUNK)], sem).wait()
    pltpu.sync_copy(rows_v, quant_hbm.at[pl.ds(wid * _BPW, _BPW)])


@jax.jit
def kernel(inputs, weight):
    flat = inputs.reshape(_N, _D)

    soft, idx, wn, loss, perp, active, usage = pl.pallas_call(
        _vq_main_body,
        grid=(_NBLK,),
        in_specs=[
            pl.BlockSpec((_BLK, _D), lambda i: (i, 0)),
            pl.BlockSpec((_K, _D), lambda i: (0, 0)),
        ],
        out_specs=(
            pl.BlockSpec((_BLK, _K), lambda i: (i, 0)),
            pl.BlockSpec((1, _BLK), lambda i: (0, i)),
            pl.BlockSpec((_K, 128), lambda i: (0, 0)),
            pl.BlockSpec(memory_space=pltpu.SMEM),
            pl.BlockSpec(memory_space=pltpu.SMEM),
            pl.BlockSpec(memory_space=pltpu.SMEM),
            pl.BlockSpec(memory_space=pltpu.SMEM),
        ),
        out_shape=(
            jax.ShapeDtypeStruct((_N, _K), jnp.float32),
            jax.ShapeDtypeStruct((1, _N), jnp.int32),
            jax.ShapeDtypeStruct((_K, 128), jnp.float32),
            jax.ShapeDtypeStruct((1, 1), jnp.float32),
            jax.ShapeDtypeStruct((1, 1), jnp.float32),
            jax.ShapeDtypeStruct((1, 1), jnp.int32),
            jax.ShapeDtypeStruct((1, 1), jnp.float32),
        ),
        scratch_shapes=[
            pltpu.SMEM((1, 1), jnp.float32),
            pltpu.VMEM((_K, _D), jnp.float32),
            pltpu.VMEM((1, _K), jnp.float32),
        ],
        compiler_params=pltpu.CompilerParams(
            dimension_semantics=("arbitrary",),
        ),
    )(flat, weight)

    sc_gather = functools.partial(
        pl.kernel,
        mesh=plsc.VectorSubcoreMesh(core_axis_name="c", subcore_axis_name="s"),
        out_type=jax.ShapeDtypeStruct((_N, 128), jnp.float32),
        scratch_types=[
            pltpu.VMEM((_NCHUNK, _CHUNK), jnp.int32),
            pltpu.VMEM((_BPW, 128), jnp.float32),
            pltpu.SemaphoreType.DMA,
        ],
    )(_sc_gather_body)
    quant = sc_gather(wn, idx.reshape(_N // _CHUNK, _CHUNK))

    return (
        loss.reshape(()),
        quant[:, :_D].reshape(inputs.shape),
        soft,
        perp.reshape(()),
        idx.reshape(_N, 1),
        active.reshape(()),
        usage.reshape(()),
    )
